# baseline (speedup 1.0000x reference)
import functools

import jax
import jax.numpy as jnp
from jax import lax
from jax.experimental import pallas as pl
from jax.experimental.pallas import tpu as pltpu
from jax.experimental.pallas import tpu_sc as plsc

_FEAT_STRIDE = 16.0
_PRE_NMS = 6000
_POST_NMS = 300
_NMS_THRESH = 0.7
_LANES = 128

_NW = 32
_CH_ROWS = 36
_CH = _CH_ROWS * _LANES
_CROWS = 49
_PAD = _CROWS * _LANES
_META = 48


def _transform_kernel(im_ref, sb_ref, dx_ref, dy_ref, dw_ref, dh_ref,
                      cx_ref, cy_ref, aw_ref, ah_ref,
                      x1_o, y1_o, x2_o, y2_o, ar_o, meta_o,
                      *, rows, pre_nms):
    b = pl.program_id(0)
    n = rows * _LANES

    ww = aw_ref[...]
    hh = ah_ref[...]
    pcx = dx_ref[0] * ww + cx_ref[...]
    pcy = dy_ref[0] * hh + cy_ref[...]
    pw = jnp.exp(dw_ref[0]) * ww
    ph = jnp.exp(dh_ref[0]) * hh
    maxw = im_ref[b, 1] - 1.0
    maxh = im_ref[b, 0] - 1.0
    x1 = jnp.minimum(jnp.maximum(pcx - 0.5 * pw, 0.0), maxw)
    y1 = jnp.minimum(jnp.maximum(pcy - 0.5 * ph, 0.0), maxh)
    x2 = jnp.minimum(jnp.maximum(pcx + 0.5 * pw, 0.0), maxw)
    y2 = jnp.minimum(jnp.maximum(pcy + 0.5 * ph, 0.0), maxh)
    x1_o[0] = x1
    y1_o[0] = y1
    x2_o[0] = x2
    y2_o[0] = y2
    ar_o[0] = (x2 - x1 + 1.0) * (y2 - y1 + 1.0)

    sbits = sb_ref[0]
    iota = (lax.broadcasted_iota(jnp.int32, (rows, _LANES), 0) * _LANES
            + lax.broadcasted_iota(jnp.int32, (rows, _LANES), 1))

    def _count_gt(t):
        return jnp.sum(jnp.where(sbits > t, 1.0, 0.0))

    kf = jnp.float32(pre_nms)

    def _bis_body(_, carry):
        lo, hi = carry
        mid = lax.div(lo + hi, 2)
        gt = _count_gt(mid) >= kf
        return (jnp.where(gt, mid, lo), jnp.where(gt, hi, mid))

    _, tbits = lax.fori_loop(0, 31, _bis_body,
                             (jnp.int32(-1), jnp.int32(1 << 30)))

    count_gt = _count_gt(tbits)
    r = kf - count_gt
    eq = sbits == tbits

    def _tie_body(_, carry):
        lo, hi = carry
        mid = lax.div(lo + hi, 2)
        cnt = jnp.sum(jnp.where(eq & (iota <= mid), 1.0, 0.0))
        ge = cnt >= r
        return (jnp.where(ge, lo, mid), jnp.where(ge, mid, hi))

    _, mcut = lax.fori_loop(0, 18, _tie_body, (jnp.int32(-1),
                                               jnp.int32(n - 1)))

    member = (sbits > tbits) | (eq & (iota <= mcut))
    meta_o[0, 0, 0] = tbits
    meta_o[0, 0, 1] = mcut
    base = jnp.float32(0.0)
    for s in range(_NW):
        meta_o[0, 0, 2 + s] = base.astype(jnp.int32)
        base = base + jnp.sum(
            jnp.where(member[s * _CH_ROWS:(s + 1) * _CH_ROWS, :], 1.0, 0.0))


def _sc_compact_kernel(sb_hbm, meta_hbm, x1_hbm, y1_hbm, x2_hbm, y2_hbm,
                       ar_hbm, x1o, y1o, x2o, y2o, aro, sbo,
                       sb_v, idx_v, meta_v, x1_v, y1_v, x2_v, y2_v, ar_v,
                       sem, *, nimg):
    wid = lax.axis_index("c") * 16 + lax.axis_index("s")
    e0 = wid * _CH
    l16 = lax.iota(jnp.int32, 16)

    for b in range(nimg):
        pltpu.sync_copy(sb_hbm.at[b, pl.ds(e0, _CH)], sb_v)
        pltpu.sync_copy(meta_hbm.at[b, 0], meta_v)
        m0 = meta_v[pl.ds(0, 16)]
        m1 = meta_v[pl.ds(16, 16)]
        m2 = meta_v[pl.ds(32, 16)]
        tbits = jnp.sum(jnp.where(l16 == 0, m0, 0))
        mcut = jnp.sum(jnp.where(l16 == 1, m0, 0))
        mpos = wid + 2
        vsel = jnp.where(mpos < 16, m0, jnp.where(mpos < 32, m1, m2))
        base = jnp.sum(jnp.where(l16 == (mpos % 16), vsel, 0))
        out0 = b * _PAD
        trash = out0 + _NW * 192 + wid

        def _row_body(r, run):
            for c8 in range(8):
                off = r * _LANES + c8 * 16
                sb = sb_v[pl.ds(off, 16)]
                gidx = e0 + off + l16
                m = (sb > tbits) | ((sb == tbits) & (gidx <= mcut))
                mi = m.astype(jnp.int32)
                inc = plsc.cumsum(mi)
                posv = out0 + base + run + (inc - mi)
                idx_v[r, pl.ds(c8 * 16, 16)] = jnp.where(m, posv, trash)
                run = run + jnp.sum(mi)
            return run

        lax.fori_loop(0, _CH_ROWS, _row_body, jnp.int32(0))

        pltpu.sync_copy(x1_hbm.at[b, pl.ds(e0, _CH)], x1_v)
        pltpu.sync_copy(y1_hbm.at[b, pl.ds(e0, _CH)], y1_v)
        pltpu.sync_copy(x2_hbm.at[b, pl.ds(e0, _CH)], x2_v)
        pltpu.sync_copy(y2_hbm.at[b, pl.ds(e0, _CH)], y2_v)
        pltpu.sync_copy(ar_hbm.at[b, pl.ds(e0, _CH)], ar_v)

        def _scat_body(j, c):
            hs = [pltpu.async_copy(src.at[pl.ds(j * _LANES, _LANES)],
                                   dst.at[idx_v.at[j]], sem)
                  for src, dst in ((x1_v, x1o), (y1_v, y1o), (x2_v, x2o),
                                   (y2_v, y2o), (ar_v, aro), (sb_v, sbo))]
            for h in hs:
                h.wait()
            return c

        lax.fori_loop(0, _CH_ROWS, _scat_body, 0)


def _nms_kernel(x1_ref, y1_ref, x2_ref, y2_ref, ar_ref, sb_ref, out_ref,
                sm_s, *, pre_nms, post_nms, thresh):
    iota = (lax.broadcasted_iota(jnp.int32, (_CROWS, _LANES), 0) * _LANES
            + lax.broadcasted_iota(jnp.int32, (_CROWS, _LANES), 1))
    sm_s[...] = jnp.where(iota < pre_nms, sb_ref[0], jnp.int32(-1))
    x1 = x1_ref[0]
    y1 = y1_ref[0]
    x2 = x2_ref[0]
    y2 = y2_ref[0]
    ar = ar_ref[0]

    def _nms_body(j, carry):
        smv = sm_s[...]
        mv = jnp.max(smv)
        sel = smv == mv
        idxv = jnp.min(jnp.where(sel, iota, jnp.int32(_PAD)))
        one = iota == idxv
        bx1 = jnp.sum(jnp.where(one, x1, 0.0))
        by1 = jnp.sum(jnp.where(one, y1, 0.0))
        bx2 = jnp.sum(jnp.where(one, x2, 0.0))
        by2 = jnp.sum(jnp.where(one, y2, 0.0))
        bar = jnp.sum(jnp.where(one, ar, 0.0))
        xx1 = jnp.maximum(bx1, x1)
        yy1 = jnp.maximum(by1, y1)
        xx2 = jnp.minimum(bx2, x2)
        yy2 = jnp.minimum(by2, y2)
        iw = jnp.maximum(0.0, xx2 - xx1 + 1.0)
        ih = jnp.maximum(0.0, yy2 - yy1 + 1.0)
        inter = iw * ih
        iou = inter / ((bar + ar) - inter)
        sm_s[...] = jnp.where(iou <= thresh, smv, jnp.int32(-1))
        valid = mv >= 0
        out_ref[0, j, 0] = jnp.where(valid, bx1, 0.0)
        out_ref[0, j, 1] = jnp.where(valid, by1, 0.0)
        out_ref[0, j, 2] = jnp.where(valid, bx2, 0.0)
        out_ref[0, j, 3] = jnp.where(valid, by2, 0.0)
        return carry

    lax.fori_loop(0, post_nms, _nms_body, 0)


def kernel(scores, bbox_deltas, im_info, anchors):
    B = scores.shape[0]
    A = anchors.shape[0]
    H = scores.shape[2]
    W = scores.shape[3]
    K = H * W
    N = K * A
    rows = N // _LANES

    sc = jnp.transpose(scores[:, A:, :, :], (0, 2, 3, 1)).reshape(B, rows,
                                                                  _LANES)
    sbits = lax.bitcast_convert_type(sc, jnp.int32)
    dl = jnp.transpose(bbox_deltas, (0, 2, 3, 1)).reshape(B, K, A, 4)
    dx = dl[..., 0].reshape(B, rows, _LANES)
    dy = dl[..., 1].reshape(B, rows, _LANES)
    dw = dl[..., 2].reshape(B, rows, _LANES)
    dh = dl[..., 3].reshape(B, rows, _LANES)

    aw = anchors[:, 2] - anchors[:, 0] + 1.0
    ah = anchors[:, 3] - anchors[:, 1] + 1.0
    acx = anchors[:, 0] + 0.5 * aw
    acy = anchors[:, 1] + 0.5 * ah
    shift_x = jnp.arange(W, dtype=jnp.float32) * _FEAT_STRIDE
    shift_y = jnp.arange(H, dtype=jnp.float32) * _FEAT_STRIDE
    sx, sy = jnp.meshgrid(shift_x, shift_y)
    cx = (sx.ravel()[:, None] + acx[None, :]).reshape(rows, _LANES)
    cy = (sy.ravel()[:, None] + acy[None, :]).reshape(rows, _LANES)
    awf = jnp.broadcast_to(aw[None, :], (K, A)).reshape(rows, _LANES)
    ahf = jnp.broadcast_to(ah[None, :], (K, A)).reshape(rows, _LANES)

    tbody = functools.partial(_transform_kernel, rows=rows,
                              pre_nms=_PRE_NMS)
    img_spec = pl.BlockSpec((1, rows, _LANES), lambda b: (b, 0, 0))
    shared_spec = pl.BlockSpec((rows, _LANES), lambda b: (0, 0))
    big = jax.ShapeDtypeStruct((B, rows, _LANES), jnp.float32)
    x1f, y1f, x2f, y2f, arf, meta = pl.pallas_call(
        tbody,
        grid=(B,),
        in_specs=[
            pl.BlockSpec(memory_space=pltpu.SMEM),
            img_spec, img_spec, img_spec, img_spec, img_spec,
            shared_spec, shared_spec, shared_spec, shared_spec,
        ],
        out_specs=[img_spec] * 5 + [
            pl.BlockSpec((1, 1, _META), lambda b: (b, 0, 0),
                         memory_space=pltpu.SMEM)],
        out_shape=[big] * 5 + [
            jax.ShapeDtypeStruct((B, 1, _META), jnp.int32)],
    )(im_info, sbits, dx, dy, dw, dh, cx, cy, awf, ahf)

    scbody = functools.partial(_sc_compact_kernel, nimg=B)
    flat = jax.ShapeDtypeStruct((B * _PAD,), jnp.float32)
    mesh = plsc.VectorSubcoreMesh(core_axis_name="c", subcore_axis_name="s")
    ch_f32 = pltpu.VMEM((_CH,), jnp.float32)
    ch_i32 = pltpu.VMEM((_CH,), jnp.int32)
    x1c, y1c, x2c, y2c, arc, sbc = pl.kernel(
        scbody,
        out_type=[flat] * 5 + [jax.ShapeDtypeStruct((B * _PAD,),
                                                    jnp.int32)],
        mesh=mesh,
        scratch_types=[ch_i32,
                       pltpu.VMEM((_CH_ROWS, _LANES), jnp.int32),
                       pltpu.VMEM((_META,), jnp.int32),
                       ch_f32, ch_f32, ch_f32, ch_f32, ch_f32,
                       pltpu.SemaphoreType.DMA],
        compiler_params=pltpu.CompilerParams(needs_layout_passes=False),
    )(sbits.reshape(B, N), meta, x1f.reshape(B, N), y1f.reshape(B, N),
      x2f.reshape(B, N), y2f.reshape(B, N), arf.reshape(B, N))

    nbody = functools.partial(_nms_kernel, pre_nms=_PRE_NMS,
                              post_nms=_POST_NMS, thresh=_NMS_THRESH)
    cimg_spec = pl.BlockSpec((1, _CROWS, _LANES), lambda b: (b, 0, 0))
    out = pl.pallas_call(
        nbody,
        grid=(B,),
        in_specs=[cimg_spec] * 6,
        out_specs=pl.BlockSpec((1, _POST_NMS, 4), lambda b: (b, 0, 0),
                               memory_space=pltpu.SMEM),
        out_shape=jax.ShapeDtypeStruct((B, _POST_NMS, 4), jnp.float32),
        scratch_shapes=[pltpu.VMEM((_CROWS, _LANES), jnp.int32)],
    )(x1c.reshape(B, _CROWS, _LANES), y1c.reshape(B, _CROWS, _LANES),
      x2c.reshape(B, _CROWS, _LANES), y2c.reshape(B, _CROWS, _LANES),
      arc.reshape(B, _CROWS, _LANES), sbc.reshape(B, _CROWS, _LANES))

    col0 = jnp.broadcast_to(
        jnp.arange(B, dtype=jnp.float32)[:, None, None], (B, _POST_NMS, 1))
    return jnp.concatenate([col0, out], axis=2)


# speedup vs baseline: 57.2557x; 57.2557x over previous
import functools

import jax
import jax.numpy as jnp
from jax import lax
from jax.experimental import pallas as pl
from jax.experimental.pallas import tpu as pltpu
from jax.experimental.pallas import tpu_sc as plsc

_FEAT_STRIDE = 16.0
_PRE_NMS = 6000
_POST_NMS = 300
_NMS_THRESH = 0.7
_LANES = 128

_NW = 32
_CH_ROWS = 36
_CH = _CH_ROWS * _LANES
_CROWS = 49
_PAD = _CROWS * _LANES
_META = 48


def _transform_kernel(im_ref, sb_ref, dx_ref, dy_ref, dw_ref, dh_ref,
                      cx_ref, cy_ref, aw_ref, ah_ref,
                      x1_o, y1_o, x2_o, y2_o, ar_o, meta_o,
                      *, rows, pre_nms):
    b = pl.program_id(0)
    n = rows * _LANES

    ww = aw_ref[...]
    hh = ah_ref[...]
    pcx = dx_ref[0] * ww + cx_ref[...]
    pcy = dy_ref[0] * hh + cy_ref[...]
    pw = jnp.exp(dw_ref[0]) * ww
    ph = jnp.exp(dh_ref[0]) * hh
    maxw = im_ref[b, 1] - 1.0
    maxh = im_ref[b, 0] - 1.0
    x1 = jnp.minimum(jnp.maximum(pcx - 0.5 * pw, 0.0), maxw)
    y1 = jnp.minimum(jnp.maximum(pcy - 0.5 * ph, 0.0), maxh)
    x2 = jnp.minimum(jnp.maximum(pcx + 0.5 * pw, 0.0), maxw)
    y2 = jnp.minimum(jnp.maximum(pcy + 0.5 * ph, 0.0), maxh)
    x1_o[0] = x1
    y1_o[0] = y1
    x2_o[0] = x2
    y2_o[0] = y2
    ar_o[0] = (x2 - x1 + 1.0) * (y2 - y1 + 1.0)

    sbits = sb_ref[0]
    iota = (lax.broadcasted_iota(jnp.int32, (rows, _LANES), 0) * _LANES
            + lax.broadcasted_iota(jnp.int32, (rows, _LANES), 1))

    def _count_gt(t):
        return jnp.sum(jnp.where(sbits > t, 1.0, 0.0))

    kf = jnp.float32(pre_nms)

    def _bis_body(_, carry):
        lo, hi = carry
        mid = lax.div(lo + hi, 2)
        gt = _count_gt(mid) >= kf
        return (jnp.where(gt, mid, lo), jnp.where(gt, hi, mid))

    _, tbits = lax.fori_loop(0, 31, _bis_body,
                             (jnp.int32(-1), jnp.int32(1 << 30)))

    count_gt = _count_gt(tbits)
    r = kf - count_gt
    eq = sbits == tbits

    def _tie_body(_, carry):
        lo, hi = carry
        mid = lax.div(lo + hi, 2)
        cnt = jnp.sum(jnp.where(eq & (iota <= mid), 1.0, 0.0))
        ge = cnt >= r
        return (jnp.where(ge, lo, mid), jnp.where(ge, mid, hi))

    _, mcut = lax.fori_loop(0, 18, _tie_body, (jnp.int32(-1),
                                               jnp.int32(n - 1)))

    member = (sbits > tbits) | (eq & (iota <= mcut))
    meta_o[0, 0, 0] = tbits
    meta_o[0, 0, 1] = mcut
    base = jnp.float32(0.0)
    for s in range(_NW):
        meta_o[0, 0, 2 + s] = base.astype(jnp.int32)
        base = base + jnp.sum(
            jnp.where(member[s * _CH_ROWS:(s + 1) * _CH_ROWS, :], 1.0, 0.0))


def _sc_compact_kernel(sb_hbm, meta_hbm, x1_hbm, y1_hbm, x2_hbm, y2_hbm,
                       ar_hbm, x1o, y1o, x2o, y2o, aro, sbo,
                       sb_v, idx_v, meta_v, x1_v, y1_v, x2_v, y2_v, ar_v,
                       x1_sh, y1_sh, x2_sh, y2_sh, ar_sh, sb_sh,
                       sem, *, nimg):
    cid = lax.axis_index("c")
    sid = lax.axis_index("s")
    l16 = lax.iota(jnp.int32, 16)

    @pl.when(cid == 0)
    def _compact():
        for b in range(nimg):
            pltpu.sync_copy(meta_hbm.at[b, 0], meta_v)
            m0 = meta_v[pl.ds(0, 16)]
            m1 = meta_v[pl.ds(16, 16)]
            m2 = meta_v[pl.ds(32, 16)]
            tbits = jnp.sum(jnp.where(l16 == 0, m0, 0))
            mcut = jnp.sum(jnp.where(l16 == 1, m0, 0))
            out0 = b * _PAD
            for k in range(2):
                chunk = sid + k * 16
                e0 = chunk * _CH
                mpos = chunk + 2
                vsel = jnp.where(mpos < 16, m0,
                                 jnp.where(mpos < 32, m1, m2))
                base = jnp.sum(jnp.where(l16 == (mpos % 16), vsel, 0))
                trash = out0 + _NW * 192 + chunk

                pltpu.sync_copy(sb_hbm.at[b, pl.ds(e0, _CH)], sb_v)

                def _row_body(r, run):
                    for c8 in range(8):
                        off = r * _LANES + c8 * 16
                        sb = sb_v[pl.ds(off, 16)]
                        gidx = e0 + off + l16
                        m = (sb > tbits) | ((sb == tbits) & (gidx <= mcut))
                        mi = m.astype(jnp.int32)
                        inc = plsc.cumsum(mi)
                        posv = out0 + base + run + (inc - mi)
                        idx_v[r, pl.ds(c8 * 16, 16)] = jnp.where(m, posv,
                                                                 trash)
                        run = run + jnp.sum(mi)
                    return run

                lax.fori_loop(0, _CH_ROWS, _row_body, jnp.int32(0))

                pltpu.sync_copy(x1_hbm.at[b, pl.ds(e0, _CH)], x1_v)
                pltpu.sync_copy(y1_hbm.at[b, pl.ds(e0, _CH)], y1_v)
                pltpu.sync_copy(x2_hbm.at[b, pl.ds(e0, _CH)], x2_v)
                pltpu.sync_copy(y2_hbm.at[b, pl.ds(e0, _CH)], y2_v)
                pltpu.sync_copy(ar_hbm.at[b, pl.ds(e0, _CH)], ar_v)

                def _scat_body(j, c):
                    hs = [pltpu.async_copy(
                        src.at[pl.ds(j * _LANES, _LANES)],
                        dst.at[idx_v.at[j]], sem)
                        for src, dst in ((x1_v, x1_sh), (y1_v, y1_sh),
                                         (x2_v, x2_sh), (y2_v, y2_sh),
                                         (ar_v, ar_sh), (sb_v, sb_sh))]
                    for h in hs:
                        h.wait()
                    return c

                lax.fori_loop(0, _CH_ROWS, _scat_body, 0)

    plsc.subcore_barrier()

    @pl.when((cid == 0) & (sid == 0))
    def _flush():
        pltpu.sync_copy(x1_sh, x1o)
        pltpu.sync_copy(y1_sh, y1o)
        pltpu.sync_copy(x2_sh, x2o)
        pltpu.sync_copy(y2_sh, y2o)
        pltpu.sync_copy(ar_sh, aro)
        pltpu.sync_copy(sb_sh, sbo)


def _nms_kernel(x1_ref, y1_ref, x2_ref, y2_ref, ar_ref, sb_ref, out_ref,
                sm_s, *, pre_nms, post_nms, thresh):
    iota = (lax.broadcasted_iota(jnp.int32, (_CROWS, _LANES), 0) * _LANES
            + lax.broadcasted_iota(jnp.int32, (_CROWS, _LANES), 1))
    sm_s[...] = jnp.where(iota < pre_nms, sb_ref[0], jnp.int32(-1))
    x1 = x1_ref[0]
    y1 = y1_ref[0]
    x2 = x2_ref[0]
    y2 = y2_ref[0]
    ar = ar_ref[0]

    def _nms_body(j, carry):
        smv = sm_s[...]
        mv = jnp.max(smv)
        sel = smv == mv
        idxv = jnp.min(jnp.where(sel, iota, jnp.int32(_PAD)))
        one = iota == idxv
        bx1 = jnp.sum(jnp.where(one, x1, 0.0))
        by1 = jnp.sum(jnp.where(one, y1, 0.0))
        bx2 = jnp.sum(jnp.where(one, x2, 0.0))
        by2 = jnp.sum(jnp.where(one, y2, 0.0))
        bar = jnp.sum(jnp.where(one, ar, 0.0))
        xx1 = jnp.maximum(bx1, x1)
        yy1 = jnp.maximum(by1, y1)
        xx2 = jnp.minimum(bx2, x2)
        yy2 = jnp.minimum(by2, y2)
        iw = jnp.maximum(0.0, xx2 - xx1 + 1.0)
        ih = jnp.maximum(0.0, yy2 - yy1 + 1.0)
        inter = iw * ih
        iou = inter / ((bar + ar) - inter)
        sm_s[...] = jnp.where(iou <= thresh, smv, jnp.int32(-1))
        valid = mv >= 0
        out_ref[0, j, 0] = jnp.where(valid, bx1, 0.0)
        out_ref[0, j, 1] = jnp.where(valid, by1, 0.0)
        out_ref[0, j, 2] = jnp.where(valid, bx2, 0.0)
        out_ref[0, j, 3] = jnp.where(valid, by2, 0.0)
        return carry

    lax.fori_loop(0, post_nms, _nms_body, 0)


def kernel(scores, bbox_deltas, im_info, anchors):
    B = scores.shape[0]
    A = anchors.shape[0]
    H = scores.shape[2]
    W = scores.shape[3]
    K = H * W
    N = K * A
    rows = N // _LANES

    sc = jnp.transpose(scores[:, A:, :, :], (0, 2, 3, 1)).reshape(B, rows,
                                                                  _LANES)
    sbits = lax.bitcast_convert_type(sc, jnp.int32)
    dl = jnp.transpose(bbox_deltas, (0, 2, 3, 1)).reshape(B, K, A, 4)
    dx = dl[..., 0].reshape(B, rows, _LANES)
    dy = dl[..., 1].reshape(B, rows, _LANES)
    dw = dl[..., 2].reshape(B, rows, _LANES)
    dh = dl[..., 3].reshape(B, rows, _LANES)

    aw = anchors[:, 2] - anchors[:, 0] + 1.0
    ah = anchors[:, 3] - anchors[:, 1] + 1.0
    acx = anchors[:, 0] + 0.5 * aw
    acy = anchors[:, 1] + 0.5 * ah
    shift_x = jnp.arange(W, dtype=jnp.float32) * _FEAT_STRIDE
    shift_y = jnp.arange(H, dtype=jnp.float32) * _FEAT_STRIDE
    sx, sy = jnp.meshgrid(shift_x, shift_y)
    cx = (sx.ravel()[:, None] + acx[None, :]).reshape(rows, _LANES)
    cy = (sy.ravel()[:, None] + acy[None, :]).reshape(rows, _LANES)
    awf = jnp.broadcast_to(aw[None, :], (K, A)).reshape(rows, _LANES)
    ahf = jnp.broadcast_to(ah[None, :], (K, A)).reshape(rows, _LANES)

    tbody = functools.partial(_transform_kernel, rows=rows,
                              pre_nms=_PRE_NMS)
    img_spec = pl.BlockSpec((1, rows, _LANES), lambda b: (b, 0, 0))
    shared_spec = pl.BlockSpec((rows, _LANES), lambda b: (0, 0))
    big = jax.ShapeDtypeStruct((B, rows, _LANES), jnp.float32)
    x1f, y1f, x2f, y2f, arf, meta = pl.pallas_call(
        tbody,
        grid=(B,),
        in_specs=[
            pl.BlockSpec(memory_space=pltpu.SMEM),
            img_spec, img_spec, img_spec, img_spec, img_spec,
            shared_spec, shared_spec, shared_spec, shared_spec,
        ],
        out_specs=[img_spec] * 5 + [
            pl.BlockSpec((1, 1, _META), lambda b: (b, 0, 0),
                         memory_space=pltpu.SMEM)],
        out_shape=[big] * 5 + [
            jax.ShapeDtypeStruct((B, 1, _META), jnp.int32)],
    )(im_info, sbits, dx, dy, dw, dh, cx, cy, awf, ahf)

    scbody = functools.partial(_sc_compact_kernel, nimg=B)
    flat = jax.ShapeDtypeStruct((B * _PAD,), jnp.float32)
    mesh = plsc.VectorSubcoreMesh(core_axis_name="c", subcore_axis_name="s")
    ch_f32 = pltpu.VMEM((_CH,), jnp.float32)
    ch_i32 = pltpu.VMEM((_CH,), jnp.int32)
    x1c, y1c, x2c, y2c, arc, sbc = pl.kernel(
        scbody,
        out_type=[flat] * 5 + [jax.ShapeDtypeStruct((B * _PAD,),
                                                    jnp.int32)],
        mesh=mesh,
        scratch_types=[ch_i32,
                       pltpu.VMEM((_CH_ROWS, _LANES), jnp.int32),
                       pltpu.VMEM((_META,), jnp.int32),
                       ch_f32, ch_f32, ch_f32, ch_f32, ch_f32,
                       pltpu.VMEM_SHARED((B * _PAD,), jnp.float32),
                       pltpu.VMEM_SHARED((B * _PAD,), jnp.float32),
                       pltpu.VMEM_SHARED((B * _PAD,), jnp.float32),
                       pltpu.VMEM_SHARED((B * _PAD,), jnp.float32),
                       pltpu.VMEM_SHARED((B * _PAD,), jnp.float32),
                       pltpu.VMEM_SHARED((B * _PAD,), jnp.int32),
                       pltpu.SemaphoreType.DMA],
        compiler_params=pltpu.CompilerParams(needs_layout_passes=False),
    )(sbits.reshape(B, N), meta, x1f.reshape(B, N), y1f.reshape(B, N),
      x2f.reshape(B, N), y2f.reshape(B, N), arf.reshape(B, N))

    nbody = functools.partial(_nms_kernel, pre_nms=_PRE_NMS,
                              post_nms=_POST_NMS, thresh=_NMS_THRESH)
    cimg_spec = pl.BlockSpec((1, _CROWS, _LANES), lambda b: (b, 0, 0))
    out = pl.pallas_call(
        nbody,
        grid=(B,),
        in_specs=[cimg_spec] * 6,
        out_specs=pl.BlockSpec((1, _POST_NMS, 4), lambda b: (b, 0, 0),
                               memory_space=pltpu.SMEM),
        out_shape=jax.ShapeDtypeStruct((B, _POST_NMS, 4), jnp.float32),
        scratch_shapes=[pltpu.VMEM((_CROWS, _LANES), jnp.int32)],
    )(x1c.reshape(B, _CROWS, _LANES), y1c.reshape(B, _CROWS, _LANES),
      x2c.reshape(B, _CROWS, _LANES), y2c.reshape(B, _CROWS, _LANES),
      arc.reshape(B, _CROWS, _LANES), sbc.reshape(B, _CROWS, _LANES))

    col0 = jnp.broadcast_to(
        jnp.arange(B, dtype=jnp.float32)[:, None, None], (B, _POST_NMS, 1))
    return jnp.concatenate([col0, out], axis=2)


# speedup vs baseline: 58.0981x; 1.0147x over previous
import functools

import jax
import jax.numpy as jnp
from jax import lax
from jax.experimental import pallas as pl
from jax.experimental.pallas import tpu as pltpu
from jax.experimental.pallas import tpu_sc as plsc

_FEAT_STRIDE = 16.0
_PRE_NMS = 6000
_POST_NMS = 300
_NMS_THRESH = 0.7
_LANES = 128

_NW = 32
_CH_ROWS = 36
_CH = _CH_ROWS * _LANES
_CROWS = 49
_PAD = _CROWS * _LANES
_META = 48


def _transform_kernel(im_ref, sb_ref, dx_ref, dy_ref, dw_ref, dh_ref,
                      cx_ref, cy_ref, aw_ref, ah_ref,
                      x1_o, y1_o, x2_o, y2_o, ar_o, meta_o,
                      *, rows, pre_nms):
    b = pl.program_id(0)
    n = rows * _LANES

    ww = aw_ref[...]
    hh = ah_ref[...]
    pcx = dx_ref[0] * ww + cx_ref[...]
    pcy = dy_ref[0] * hh + cy_ref[...]
    pw = jnp.exp(dw_ref[0]) * ww
    ph = jnp.exp(dh_ref[0]) * hh
    maxw = im_ref[b, 1] - 1.0
    maxh = im_ref[b, 0] - 1.0
    x1 = jnp.minimum(jnp.maximum(pcx - 0.5 * pw, 0.0), maxw)
    y1 = jnp.minimum(jnp.maximum(pcy - 0.5 * ph, 0.0), maxh)
    x2 = jnp.minimum(jnp.maximum(pcx + 0.5 * pw, 0.0), maxw)
    y2 = jnp.minimum(jnp.maximum(pcy + 0.5 * ph, 0.0), maxh)
    x1_o[0] = x1
    y1_o[0] = y1
    x2_o[0] = x2
    y2_o[0] = y2
    ar_o[0] = (x2 - x1 + 1.0) * (y2 - y1 + 1.0)

    sbits = sb_ref[0]
    iota = (lax.broadcasted_iota(jnp.int32, (rows, _LANES), 0) * _LANES
            + lax.broadcasted_iota(jnp.int32, (rows, _LANES), 1))

    def _count_gt(t):
        return jnp.sum(jnp.where(sbits > t, 1.0, 0.0))

    kf = jnp.float32(pre_nms)

    def _bis_body(_, carry):
        lo, hi = carry
        mid = lax.div(lo + hi, 2)
        gt = _count_gt(mid) >= kf
        return (jnp.where(gt, mid, lo), jnp.where(gt, hi, mid))

    _, tbits = lax.fori_loop(0, 31, _bis_body,
                             (jnp.int32(-1), jnp.int32(1 << 30)))

    count_gt = _count_gt(tbits)
    r = kf - count_gt
    eq = sbits == tbits

    def _tie_body(_, carry):
        lo, hi = carry
        mid = lax.div(lo + hi, 2)
        cnt = jnp.sum(jnp.where(eq & (iota <= mid), 1.0, 0.0))
        ge = cnt >= r
        return (jnp.where(ge, lo, mid), jnp.where(ge, mid, hi))

    _, mcut = lax.fori_loop(0, 18, _tie_body, (jnp.int32(-1),
                                               jnp.int32(n - 1)))

    member = (sbits > tbits) | (eq & (iota <= mcut))
    meta_o[0, 0, 0] = tbits
    meta_o[0, 0, 1] = mcut
    base = jnp.float32(0.0)
    for s in range(_NW):
        meta_o[0, 0, 2 + s] = base.astype(jnp.int32)
        base = base + jnp.sum(
            jnp.where(member[s * _CH_ROWS:(s + 1) * _CH_ROWS, :], 1.0, 0.0))


def _sc_compact_kernel(sb_hbm, meta_hbm, x1_hbm, y1_hbm, x2_hbm, y2_hbm,
                       ar_hbm, x1o, y1o, x2o, y2o, aro, sbo,
                       sb_v, idx_v, meta_v, x1_v, y1_v, x2_v, y2_v, ar_v,
                       x1_sh, y1_sh, x2_sh, y2_sh, ar_sh, sb_sh,
                       sem, sem2, *, nimg):
    cid = lax.axis_index("c")
    sid = lax.axis_index("s")
    l16 = lax.iota(jnp.int32, 16)

    @pl.when(cid == 0)
    def _compact():
        for b in range(nimg):
            pltpu.sync_copy(meta_hbm.at[b, 0], meta_v)
            m0 = meta_v[pl.ds(0, 16)]
            m1 = meta_v[pl.ds(16, 16)]
            m2 = meta_v[pl.ds(32, 16)]
            tbits = jnp.sum(jnp.where(l16 == 0, m0, 0))
            mcut = jnp.sum(jnp.where(l16 == 1, m0, 0))
            out0 = b * _PAD
            for k in range(2):
                chunk = sid + k * 16
                e0 = chunk * _CH
                mpos = chunk + 2
                vsel = jnp.where(mpos < 16, m0,
                                 jnp.where(mpos < 32, m1, m2))
                base = jnp.sum(jnp.where(l16 == (mpos % 16), vsel, 0))
                trash = out0 + _NW * 192 + chunk

                stage = [pltpu.async_copy(src.at[b, pl.ds(e0, _CH)], dst,
                                          sem2)
                         for src, dst in ((x1_hbm, x1_v), (y1_hbm, y1_v),
                                          (x2_hbm, x2_v), (y2_hbm, y2_v),
                                          (ar_hbm, ar_v))]
                pltpu.sync_copy(sb_hbm.at[b, pl.ds(e0, _CH)], sb_v)

                def _row_body(r, run):
                    for c8 in range(8):
                        off = r * _LANES + c8 * 16
                        sb = sb_v[pl.ds(off, 16)]
                        gidx = e0 + off + l16
                        m = (sb > tbits) | ((sb == tbits) & (gidx <= mcut))
                        mi = m.astype(jnp.int32)
                        inc = plsc.cumsum(mi)
                        posv = out0 + base + run + (inc - mi)
                        idx_v[r, pl.ds(c8 * 16, 16)] = jnp.where(m, posv,
                                                                 trash)
                        run = run + jnp.sum(mi)
                    return run

                lax.fori_loop(0, _CH_ROWS, _row_body, jnp.int32(0))

                for h in stage:
                    h.wait()

                def _scat_body(j, c):
                    for src, dst in ((x1_v, x1_sh), (y1_v, y1_sh),
                                     (x2_v, x2_sh), (y2_v, y2_sh),
                                     (ar_v, ar_sh), (sb_v, sb_sh)):
                        pltpu.async_copy(
                            src.at[pl.ds(j * _LANES, _LANES)],
                            dst.at[idx_v.at[j]], sem)
                    return c

                lax.fori_loop(0, _CH_ROWS, _scat_body, 0)
                for _ in range(6):
                    pltpu.make_async_copy(
                        sb_hbm.at[b, pl.ds(e0, _CH)], sb_v, sem).wait()

    plsc.subcore_barrier()

    @pl.when((cid == 0) & (sid == 0))
    def _flush():
        pltpu.sync_copy(x1_sh, x1o)
        pltpu.sync_copy(y1_sh, y1o)
        pltpu.sync_copy(x2_sh, x2o)
        pltpu.sync_copy(y2_sh, y2o)
        pltpu.sync_copy(ar_sh, aro)
        pltpu.sync_copy(sb_sh, sbo)


def _nms_kernel(x1_ref, y1_ref, x2_ref, y2_ref, ar_ref, sb_ref, out_ref,
                sm_s, *, pre_nms, post_nms, thresh):
    iota = (lax.broadcasted_iota(jnp.int32, (_CROWS, _LANES), 0) * _LANES
            + lax.broadcasted_iota(jnp.int32, (_CROWS, _LANES), 1))
    sm_s[...] = jnp.where(iota < pre_nms, sb_ref[0], jnp.int32(-1))
    x1 = x1_ref[0]
    y1 = y1_ref[0]
    x2 = x2_ref[0]
    y2 = y2_ref[0]
    ar = ar_ref[0]

    def _nms_body(j, carry):
        smv = sm_s[...]
        mv = jnp.max(smv)
        sel = smv == mv
        idxv = jnp.min(jnp.where(sel, iota, jnp.int32(_PAD)))
        one = iota == idxv
        bx1 = jnp.sum(jnp.where(one, x1, 0.0))
        by1 = jnp.sum(jnp.where(one, y1, 0.0))
        bx2 = jnp.sum(jnp.where(one, x2, 0.0))
        by2 = jnp.sum(jnp.where(one, y2, 0.0))
        bar = jnp.sum(jnp.where(one, ar, 0.0))
        xx1 = jnp.maximum(bx1, x1)
        yy1 = jnp.maximum(by1, y1)
        xx2 = jnp.minimum(bx2, x2)
        yy2 = jnp.minimum(by2, y2)
        iw = jnp.maximum(0.0, xx2 - xx1 + 1.0)
        ih = jnp.maximum(0.0, yy2 - yy1 + 1.0)
        inter = iw * ih
        iou = inter / ((bar + ar) - inter)
        sm_s[...] = jnp.where(iou <= thresh, smv, jnp.int32(-1))
        valid = mv >= 0
        out_ref[0, j, 0] = jnp.where(valid, bx1, 0.0)
        out_ref[0, j, 1] = jnp.where(valid, by1, 0.0)
        out_ref[0, j, 2] = jnp.where(valid, bx2, 0.0)
        out_ref[0, j, 3] = jnp.where(valid, by2, 0.0)
        return carry

    lax.fori_loop(0, post_nms, _nms_body, 0)


def kernel(scores, bbox_deltas, im_info, anchors):
    B = scores.shape[0]
    A = anchors.shape[0]
    H = scores.shape[2]
    W = scores.shape[3]
    K = H * W
    N = K * A
    rows = N // _LANES

    sc = jnp.transpose(scores[:, A:, :, :], (0, 2, 3, 1)).reshape(B, rows,
                                                                  _LANES)
    sbits = lax.bitcast_convert_type(sc, jnp.int32)
    dl = jnp.transpose(bbox_deltas, (0, 2, 3, 1)).reshape(B, K, A, 4)
    dx = dl[..., 0].reshape(B, rows, _LANES)
    dy = dl[..., 1].reshape(B, rows, _LANES)
    dw = dl[..., 2].reshape(B, rows, _LANES)
    dh = dl[..., 3].reshape(B, rows, _LANES)

    aw = anchors[:, 2] - anchors[:, 0] + 1.0
    ah = anchors[:, 3] - anchors[:, 1] + 1.0
    acx = anchors[:, 0] + 0.5 * aw
    acy = anchors[:, 1] + 0.5 * ah
    shift_x = jnp.arange(W, dtype=jnp.float32) * _FEAT_STRIDE
    shift_y = jnp.arange(H, dtype=jnp.float32) * _FEAT_STRIDE
    sx, sy = jnp.meshgrid(shift_x, shift_y)
    cx = (sx.ravel()[:, None] + acx[None, :]).reshape(rows, _LANES)
    cy = (sy.ravel()[:, None] + acy[None, :]).reshape(rows, _LANES)
    awf = jnp.broadcast_to(aw[None, :], (K, A)).reshape(rows, _LANES)
    ahf = jnp.broadcast_to(ah[None, :], (K, A)).reshape(rows, _LANES)

    tbody = functools.partial(_transform_kernel, rows=rows,
                              pre_nms=_PRE_NMS)
    img_spec = pl.BlockSpec((1, rows, _LANES), lambda b: (b, 0, 0))
    shared_spec = pl.BlockSpec((rows, _LANES), lambda b: (0, 0))
    big = jax.ShapeDtypeStruct((B, rows, _LANES), jnp.float32)
    x1f, y1f, x2f, y2f, arf, meta = pl.pallas_call(
        tbody,
        grid=(B,),
        in_specs=[
            pl.BlockSpec(memory_space=pltpu.SMEM),
            img_spec, img_spec, img_spec, img_spec, img_spec,
            shared_spec, shared_spec, shared_spec, shared_spec,
        ],
        out_specs=[img_spec] * 5 + [
            pl.BlockSpec((1, 1, _META), lambda b: (b, 0, 0),
                         memory_space=pltpu.SMEM)],
        out_shape=[big] * 5 + [
            jax.ShapeDtypeStruct((B, 1, _META), jnp.int32)],
    )(im_info, sbits, dx, dy, dw, dh, cx, cy, awf, ahf)

    scbody = functools.partial(_sc_compact_kernel, nimg=B)
    flat = jax.ShapeDtypeStruct((B * _PAD,), jnp.float32)
    mesh = plsc.VectorSubcoreMesh(core_axis_name="c", subcore_axis_name="s")
    ch_f32 = pltpu.VMEM((_CH,), jnp.float32)
    ch_i32 = pltpu.VMEM((_CH,), jnp.int32)
    x1c, y1c, x2c, y2c, arc, sbc = pl.kernel(
        scbody,
        out_type=[flat] * 5 + [jax.ShapeDtypeStruct((B * _PAD,),
                                                    jnp.int32)],
        mesh=mesh,
        scratch_types=[ch_i32,
                       pltpu.VMEM((_CH_ROWS, _LANES), jnp.int32),
                       pltpu.VMEM((_META,), jnp.int32),
                       ch_f32, ch_f32, ch_f32, ch_f32, ch_f32,
                       pltpu.VMEM_SHARED((B * _PAD,), jnp.float32),
                       pltpu.VMEM_SHARED((B * _PAD,), jnp.float32),
                       pltpu.VMEM_SHARED((B * _PAD,), jnp.float32),
                       pltpu.VMEM_SHARED((B * _PAD,), jnp.float32),
                       pltpu.VMEM_SHARED((B * _PAD,), jnp.float32),
                       pltpu.VMEM_SHARED((B * _PAD,), jnp.int32),
                       pltpu.SemaphoreType.DMA, pltpu.SemaphoreType.DMA],
        compiler_params=pltpu.CompilerParams(needs_layout_passes=False),
    )(sbits.reshape(B, N), meta, x1f.reshape(B, N), y1f.reshape(B, N),
      x2f.reshape(B, N), y2f.reshape(B, N), arf.reshape(B, N))

    nbody = functools.partial(_nms_kernel, pre_nms=_PRE_NMS,
                              post_nms=_POST_NMS, thresh=_NMS_THRESH)
    cimg_spec = pl.BlockSpec((1, _CROWS, _LANES), lambda b: (b, 0, 0))
    out = pl.pallas_call(
        nbody,
        grid=(B,),
        in_specs=[cimg_spec] * 6,
        out_specs=pl.BlockSpec((1, _POST_NMS, 4), lambda b: (b, 0, 0),
                               memory_space=pltpu.SMEM),
        out_shape=jax.ShapeDtypeStruct((B, _POST_NMS, 4), jnp.float32),
        scratch_shapes=[pltpu.VMEM((_CROWS, _LANES), jnp.int32)],
    )(x1c.reshape(B, _CROWS, _LANES), y1c.reshape(B, _CROWS, _LANES),
      x2c.reshape(B, _CROWS, _LANES), y2c.reshape(B, _CROWS, _LANES),
      arc.reshape(B, _CROWS, _LANES), sbc.reshape(B, _CROWS, _LANES))

    col0 = jnp.broadcast_to(
        jnp.arange(B, dtype=jnp.float32)[:, None, None], (B, _POST_NMS, 1))
    return jnp.concatenate([col0, out], axis=2)


# speedup vs baseline: 62.4826x; 1.0755x over previous
import functools

import jax
import jax.numpy as jnp
from jax import lax
from jax.experimental import pallas as pl
from jax.experimental.pallas import tpu as pltpu
from jax.experimental.pallas import tpu_sc as plsc

_FEAT_STRIDE = 16.0
_PRE_NMS = 6000
_POST_NMS = 300
_NMS_THRESH = 0.7
_LANES = 128

_NW = 32
_CH_ROWS = 36
_CH = _CH_ROWS * _LANES
_CROWS = 49
_PAD = _CROWS * _LANES
_META = 48


def _transform_kernel(im_ref, sb_ref, dx_ref, dy_ref, dw_ref, dh_ref,
                      cx_ref, cy_ref, aw_ref, ah_ref,
                      x1_o, y1_o, x2_o, y2_o, ar_o, meta_o,
                      *, rows, pre_nms):
    b = pl.program_id(0)
    n = rows * _LANES

    ww = aw_ref[...]
    hh = ah_ref[...]
    pcx = dx_ref[0] * ww + cx_ref[...]
    pcy = dy_ref[0] * hh + cy_ref[...]
    pw = jnp.exp(dw_ref[0]) * ww
    ph = jnp.exp(dh_ref[0]) * hh
    maxw = im_ref[b, 1] - 1.0
    maxh = im_ref[b, 0] - 1.0
    x1 = jnp.minimum(jnp.maximum(pcx - 0.5 * pw, 0.0), maxw)
    y1 = jnp.minimum(jnp.maximum(pcy - 0.5 * ph, 0.0), maxh)
    x2 = jnp.minimum(jnp.maximum(pcx + 0.5 * pw, 0.0), maxw)
    y2 = jnp.minimum(jnp.maximum(pcy + 0.5 * ph, 0.0), maxh)
    x1_o[0] = x1
    y1_o[0] = y1
    x2_o[0] = x2
    y2_o[0] = y2
    ar_o[0] = (x2 - x1 + 1.0) * (y2 - y1 + 1.0)

    sbits = sb_ref[0]
    iota = (lax.broadcasted_iota(jnp.int32, (rows, _LANES), 0) * _LANES
            + lax.broadcasted_iota(jnp.int32, (rows, _LANES), 1))

    def _count_gt(t):
        return jnp.sum(jnp.where(sbits > t, 1.0, 0.0))

    kf = jnp.float32(pre_nms)

    def _bis_body(_, carry):
        lo, hi = carry
        mid = lax.div(lo + hi, 2)
        gt = _count_gt(mid) >= kf
        return (jnp.where(gt, mid, lo), jnp.where(gt, hi, mid))

    _, tbits = lax.fori_loop(0, 31, _bis_body,
                             (jnp.int32(-1), jnp.int32(1 << 30)))

    count_gt = _count_gt(tbits)
    r = kf - count_gt
    eq = sbits == tbits

    def _tie_body(_, carry):
        lo, hi = carry
        mid = lax.div(lo + hi, 2)
        cnt = jnp.sum(jnp.where(eq & (iota <= mid), 1.0, 0.0))
        ge = cnt >= r
        return (jnp.where(ge, lo, mid), jnp.where(ge, mid, hi))

    _, mcut = lax.fori_loop(0, 18, _tie_body, (jnp.int32(-1),
                                               jnp.int32(n - 1)))

    member = (sbits > tbits) | (eq & (iota <= mcut))
    meta_o[0, 0, 0] = tbits
    meta_o[0, 0, 1] = mcut
    base = jnp.float32(0.0)
    for s in range(_NW):
        meta_o[0, 0, 2 + s] = base.astype(jnp.int32)
        base = base + jnp.sum(
            jnp.where(member[s * _CH_ROWS:(s + 1) * _CH_ROWS, :], 1.0, 0.0))


def _sc_compact_kernel(sb_hbm, meta_hbm, x1_hbm, y1_hbm, x2_hbm, y2_hbm,
                       ar_hbm, x1o, y1o, x2o, y2o, aro, sbo,
                       sb_v, idx_v, meta_v, x1_v, y1_v, x2_v, y2_v, ar_v,
                       x1_sh, y1_sh, x2_sh, y2_sh, ar_sh, sb_sh,
                       sem, sem2, *, nimg):
    cid = lax.axis_index("c")
    sid = lax.axis_index("s")
    l16 = lax.iota(jnp.int32, 16)

    @pl.when(cid == 0)
    def _compact():
        for b in range(nimg):
            pltpu.sync_copy(meta_hbm.at[b, 0], meta_v)
            m0 = meta_v[pl.ds(0, 16)]
            m1 = meta_v[pl.ds(16, 16)]
            m2 = meta_v[pl.ds(32, 16)]
            tbits = jnp.sum(jnp.where(l16 == 0, m0, 0))
            mcut = jnp.sum(jnp.where(l16 == 1, m0, 0))
            out0 = b * _PAD
            for k in range(2):
                chunk = sid + k * 16
                e0 = chunk * _CH
                mpos = chunk + 2
                vsel = jnp.where(mpos < 16, m0,
                                 jnp.where(mpos < 32, m1, m2))
                base = jnp.sum(jnp.where(l16 == (mpos % 16), vsel, 0))
                trash = out0 + _NW * 192 + chunk

                stage = [pltpu.async_copy(src.at[b, pl.ds(e0, _CH)], dst,
                                          sem2)
                         for src, dst in ((x1_hbm, x1_v), (y1_hbm, y1_v),
                                          (x2_hbm, x2_v), (y2_hbm, y2_v),
                                          (ar_hbm, ar_v))]
                pltpu.sync_copy(sb_hbm.at[b, pl.ds(e0, _CH)], sb_v)

                def _row_body(r, run):
                    for c8 in range(8):
                        off = r * _LANES + c8 * 16
                        sb = sb_v[pl.ds(off, 16)]
                        gidx = e0 + off + l16
                        m = (sb > tbits) | ((sb == tbits) & (gidx <= mcut))
                        mi = m.astype(jnp.int32)
                        inc = plsc.cumsum(mi)
                        posv = out0 + base + run + (inc - mi)
                        idx_v[r, pl.ds(c8 * 16, 16)] = jnp.where(m, posv,
                                                                 trash)
                        run = run + jnp.sum(mi)
                    return run

                lax.fori_loop(0, _CH_ROWS, _row_body, jnp.int32(0))

                for h in stage:
                    h.wait()

                def _scat_body(j, c):
                    for src, dst in ((x1_v, x1_sh), (y1_v, y1_sh),
                                     (x2_v, x2_sh), (y2_v, y2_sh),
                                     (ar_v, ar_sh), (sb_v, sb_sh)):
                        pltpu.async_copy(
                            src.at[pl.ds(j * _LANES, _LANES)],
                            dst.at[idx_v.at[j]], sem)
                    return c

                lax.fori_loop(0, _CH_ROWS, _scat_body, 0)
                for _ in range(6):
                    pltpu.make_async_copy(
                        sb_hbm.at[b, pl.ds(e0, _CH)], sb_v, sem).wait()

    plsc.subcore_barrier()

    @pl.when((cid == 0) & (sid == 0))
    def _flush():
        pltpu.sync_copy(x1_sh, x1o)
        pltpu.sync_copy(y1_sh, y1o)
        pltpu.sync_copy(x2_sh, x2o)
        pltpu.sync_copy(y2_sh, y2o)
        pltpu.sync_copy(ar_sh, aro)
        pltpu.sync_copy(sb_sh, sbo)


def _nms_kernel(x1_ref, y1_ref, x2_ref, y2_ref, ar_ref, sb_ref, out_ref,
                sm_s, *, pre_nms, post_nms, thresh, nimg):
    iota = (lax.broadcasted_iota(jnp.int32, (_CROWS, _LANES), 0) * _LANES
            + lax.broadcasted_iota(jnp.int32, (_CROWS, _LANES), 1))
    for b in range(nimg):
        sm_s[b] = jnp.where(iota < pre_nms, sb_ref[b], jnp.int32(-1))

    def _nms_body(j, carry):
        for b in range(nimg):
            x1 = x1_ref[b]
            y1 = y1_ref[b]
            x2 = x2_ref[b]
            y2 = y2_ref[b]
            ar = ar_ref[b]
            smv = sm_s[b]
            mv = jnp.max(smv)
            sel = smv == mv
            idxv = jnp.min(jnp.where(sel, iota, jnp.int32(_PAD)))
            one = iota == idxv
            bx1 = jnp.sum(jnp.where(one, x1, 0.0))
            by1 = jnp.sum(jnp.where(one, y1, 0.0))
            bx2 = jnp.sum(jnp.where(one, x2, 0.0))
            by2 = jnp.sum(jnp.where(one, y2, 0.0))
            bar = jnp.sum(jnp.where(one, ar, 0.0))
            xx1 = jnp.maximum(bx1, x1)
            yy1 = jnp.maximum(by1, y1)
            xx2 = jnp.minimum(bx2, x2)
            yy2 = jnp.minimum(by2, y2)
            iw = jnp.maximum(0.0, xx2 - xx1 + 1.0)
            ih = jnp.maximum(0.0, yy2 - yy1 + 1.0)
            inter = iw * ih
            iou = inter / ((bar + ar) - inter)
            sm_s[b] = jnp.where(iou <= thresh, smv, jnp.int32(-1))
            valid = mv >= 0
            out_ref[b, j, 0] = jnp.where(valid, bx1, 0.0)
            out_ref[b, j, 1] = jnp.where(valid, by1, 0.0)
            out_ref[b, j, 2] = jnp.where(valid, bx2, 0.0)
            out_ref[b, j, 3] = jnp.where(valid, by2, 0.0)
        return carry

    lax.fori_loop(0, post_nms, _nms_body, 0)


def kernel(scores, bbox_deltas, im_info, anchors):
    B = scores.shape[0]
    A = anchors.shape[0]
    H = scores.shape[2]
    W = scores.shape[3]
    K = H * W
    N = K * A
    rows = N // _LANES

    sc = jnp.transpose(scores[:, A:, :, :], (0, 2, 3, 1)).reshape(B, rows,
                                                                  _LANES)
    sbits = lax.bitcast_convert_type(sc, jnp.int32)
    dl = jnp.transpose(bbox_deltas, (0, 2, 3, 1)).reshape(B, K, A, 4)
    dx = dl[..., 0].reshape(B, rows, _LANES)
    dy = dl[..., 1].reshape(B, rows, _LANES)
    dw = dl[..., 2].reshape(B, rows, _LANES)
    dh = dl[..., 3].reshape(B, rows, _LANES)

    aw = anchors[:, 2] - anchors[:, 0] + 1.0
    ah = anchors[:, 3] - anchors[:, 1] + 1.0
    acx = anchors[:, 0] + 0.5 * aw
    acy = anchors[:, 1] + 0.5 * ah
    shift_x = jnp.arange(W, dtype=jnp.float32) * _FEAT_STRIDE
    shift_y = jnp.arange(H, dtype=jnp.float32) * _FEAT_STRIDE
    sx, sy = jnp.meshgrid(shift_x, shift_y)
    cx = (sx.ravel()[:, None] + acx[None, :]).reshape(rows, _LANES)
    cy = (sy.ravel()[:, None] + acy[None, :]).reshape(rows, _LANES)
    awf = jnp.broadcast_to(aw[None, :], (K, A)).reshape(rows, _LANES)
    ahf = jnp.broadcast_to(ah[None, :], (K, A)).reshape(rows, _LANES)

    tbody = functools.partial(_transform_kernel, rows=rows,
                              pre_nms=_PRE_NMS)
    img_spec = pl.BlockSpec((1, rows, _LANES), lambda b: (b, 0, 0))
    shared_spec = pl.BlockSpec((rows, _LANES), lambda b: (0, 0))
    big = jax.ShapeDtypeStruct((B, rows, _LANES), jnp.float32)
    x1f, y1f, x2f, y2f, arf, meta = pl.pallas_call(
        tbody,
        grid=(B,),
        in_specs=[
            pl.BlockSpec(memory_space=pltpu.SMEM),
            img_spec, img_spec, img_spec, img_spec, img_spec,
            shared_spec, shared_spec, shared_spec, shared_spec,
        ],
        out_specs=[img_spec] * 5 + [
            pl.BlockSpec((1, 1, _META), lambda b: (b, 0, 0),
                         memory_space=pltpu.SMEM)],
        out_shape=[big] * 5 + [
            jax.ShapeDtypeStruct((B, 1, _META), jnp.int32)],
    )(im_info, sbits, dx, dy, dw, dh, cx, cy, awf, ahf)

    scbody = functools.partial(_sc_compact_kernel, nimg=B)
    flat = jax.ShapeDtypeStruct((B * _PAD,), jnp.float32)
    mesh = plsc.VectorSubcoreMesh(core_axis_name="c", subcore_axis_name="s")
    ch_f32 = pltpu.VMEM((_CH,), jnp.float32)
    ch_i32 = pltpu.VMEM((_CH,), jnp.int32)
    x1c, y1c, x2c, y2c, arc, sbc = pl.kernel(
        scbody,
        out_type=[flat] * 5 + [jax.ShapeDtypeStruct((B * _PAD,),
                                                    jnp.int32)],
        mesh=mesh,
        scratch_types=[ch_i32,
                       pltpu.VMEM((_CH_ROWS, _LANES), jnp.int32),
                       pltpu.VMEM((_META,), jnp.int32),
                       ch_f32, ch_f32, ch_f32, ch_f32, ch_f32,
                       pltpu.VMEM_SHARED((B * _PAD,), jnp.float32),
                       pltpu.VMEM_SHARED((B * _PAD,), jnp.float32),
                       pltpu.VMEM_SHARED((B * _PAD,), jnp.float32),
                       pltpu.VMEM_SHARED((B * _PAD,), jnp.float32),
                       pltpu.VMEM_SHARED((B * _PAD,), jnp.float32),
                       pltpu.VMEM_SHARED((B * _PAD,), jnp.int32),
                       pltpu.SemaphoreType.DMA, pltpu.SemaphoreType.DMA],
        compiler_params=pltpu.CompilerParams(needs_layout_passes=False),
    )(sbits.reshape(B, N), meta, x1f.reshape(B, N), y1f.reshape(B, N),
      x2f.reshape(B, N), y2f.reshape(B, N), arf.reshape(B, N))

    nbody = functools.partial(_nms_kernel, pre_nms=_PRE_NMS,
                              post_nms=_POST_NMS, thresh=_NMS_THRESH,
                              nimg=B)
    out = pl.pallas_call(
        nbody,
        out_specs=pl.BlockSpec(memory_space=pltpu.SMEM),
        out_shape=jax.ShapeDtypeStruct((B, _POST_NMS, 4), jnp.float32),
        scratch_shapes=[pltpu.VMEM((B, _CROWS, _LANES), jnp.int32)],
    )(x1c.reshape(B, _CROWS, _LANES), y1c.reshape(B, _CROWS, _LANES),
      x2c.reshape(B, _CROWS, _LANES), y2c.reshape(B, _CROWS, _LANES),
      arc.reshape(B, _CROWS, _LANES), sbc.reshape(B, _CROWS, _LANES))

    col0 = jnp.broadcast_to(
        jnp.arange(B, dtype=jnp.float32)[:, None, None], (B, _POST_NMS, 1))
    return jnp.concatenate([col0, out], axis=2)
